# baseline (speedup 1.0000x reference)
import functools

import jax
import jax.numpy as jnp
from jax import lax
from jax.experimental import pallas as pl
from jax.experimental.pallas import tpu as pltpu
from jax.experimental.pallas import tpu_sc as plsc

B, L, V = 128, 50, 10000
N_ITEMS = B * L
NC, NS = 2, 16
NW = NC * NS
PER_W = N_ITEMS // NW
CHUNKS = (PER_W + 15) // 16
PAD = CHUNKS * 16


def _lane_sum(v, lanes):
    for d in (8, 4, 2, 1):
        v = v + v.at[lanes ^ d].get(mode="promise_in_bounds")
    return v


def _body(props_ref, sw_ref, tgt_ref, adv_ref, out_ref,
          sw_v, tg_v, adv_v, vals_v, part_v, red_v, out_v, shared, sem):
    cid = lax.axis_index("c")
    sid = lax.axis_index("s")
    wid = sid * NC + cid
    base = wid * PER_W

    pltpu.sync_copy(sw_ref.at[pl.ds(base, PER_W)], sw_v.at[pl.ds(0, PER_W)])
    pltpu.sync_copy(tgt_ref.at[pl.ds(base, PER_W)], tg_v.at[pl.ds(0, PER_W)])
    pltpu.sync_copy(adv_ref, adv_v)

    lanes = lax.iota(jnp.int32, 16)

    achunks = [adv_v[pl.ds(i * 16, 16)] for i in range(B // 16)]
    s = jnp.zeros((16,), jnp.float32)
    for c in achunks:
        s = s + c
    mean = _lane_sum(s, lanes) / jnp.float32(B)
    q = jnp.zeros((16,), jnp.float32)
    for c in achunks:
        d = c - mean
        q = q + d * d
    x = _lane_sum(q, lanes) / jnp.float32(B - 1)
    seed = jnp.int32(0x5F3759DF) - lax.shift_right_logical(
        lax.bitcast_convert_type(x, jnp.int32), 1)
    y = lax.bitcast_convert_type(seed, jnp.float32)
    for _ in range(4):
        y = y * (jnp.float32(1.5) - jnp.float32(0.5) * x * y * y)
    rstd = jnp.minimum(y, jnp.float32(1e8))

    start = wid * (PER_W // L)
    cidx = lax.div(start, jnp.int32(16))
    pos = lax.rem(start, jnp.int32(16))
    avec = adv_v[pl.ds(cidx * 16, 16)]
    perm = jnp.minimum(pos + lanes, jnp.int32(15))
    adv4n = (avec.at[perm].get(mode="promise_in_bounds") - mean) * rstd


    copies = []
    for k in range(CHUNKS):
        off = k * 16
        it = base + off + lanes
        sw = sw_v[pl.ds(off, 16)]
        if off + 16 > PER_W:
            tail = lanes < (PER_W - off)
            it = jnp.where(tail, it, 0)
            sw = jnp.where(tail, sw, 0)
        flat = it * V + sw
        copies.append(pltpu.async_copy(props_ref.at[flat], vals_v.at[k], sem))
    for c in copies:
        c.wait()


    nacc = jnp.zeros((16,), jnp.float32)
    dacc = jnp.zeros((16,), jnp.float32)
    for k in range(CHUNKS):
        off = k * 16
        valid = tg_v[pl.ds(off, 16)] > 0
        if off + 16 > PER_W:
            valid = valid & (lanes < (PER_W - off))
        m = jnp.where(valid, jnp.float32(1.0), jnp.float32(0.0))
        jvec = lax.div(jnp.int32(off) + lanes, jnp.int32(L))
        a = adv4n.at[jvec].get(mode="promise_in_bounds")
        nacc = nacc - vals_v[k] * m * a
        dacc = dacc + m

    n_s = _lane_sum(nacc, lanes)
    d_s = _lane_sum(dacc, lanes)
    part_v[...] = jnp.where(lanes == 0, n_s,
                            jnp.where(lanes == 1, d_s, jnp.float32(0.0)))
    pltpu.sync_copy(part_v, shared.at[pl.ds(sid * 16, 16)])
    plsc.subcore_barrier()

    @pl.when(sid == 0)
    def _():
        pltpu.sync_copy(shared, red_v)
        tot = jnp.zeros((16,), jnp.float32)
        for i in range(NS):
            tot = tot + red_v[pl.ds(i * 16, 16)]
        out_v[...] = tot
        pltpu.sync_copy(out_v, out_ref.at[pl.ds(cid * 16, 16)])


_sc_call = functools.partial(
    pl.kernel,
    mesh=plsc.VectorSubcoreMesh(core_axis_name="c", subcore_axis_name="s"),
    out_type=jax.ShapeDtypeStruct((NC * 16,), jnp.float32),
    scratch_types=[
        pltpu.VMEM((PAD,), jnp.int32),
        pltpu.VMEM((PAD,), jnp.int32),
        pltpu.VMEM((B,), jnp.float32),
        pltpu.VMEM((CHUNKS, 16), jnp.float32),
        pltpu.VMEM((16,), jnp.float32),
        pltpu.VMEM((NS * 16,), jnp.float32),
        pltpu.VMEM((16,), jnp.float32),
        pltpu.VMEM_SHARED((NS * 16,), jnp.float32),
        pltpu.SemaphoreType.DMA,
    ],
)(_body)


def kernel(props, s_words, tgt, advantage):
    pf = props.reshape(-1)
    sw = s_words.astype(jnp.int32).reshape(-1)
    tg = tgt.astype(jnp.int32).reshape(-1)
    adv = advantage.astype(jnp.float32)
    out = _sc_call(pf, sw, tg, adv)
    return (out[0] + out[16]) / (out[1] + out[17])


# speedup vs baseline: 9.9670x; 9.9670x over previous
import functools

import jax
import jax.numpy as jnp
from jax import lax
from jax.experimental import pallas as pl
from jax.experimental.pallas import tpu as pltpu
from jax.experimental.pallas import tpu_sc as plsc

B, L, V = 128, 50, 10000
N_ITEMS = B * L
NC, NS = 2, 16
NW = NC * NS
PER_W = N_ITEMS // NW
CHUNKS = (PER_W + 15) // 16
PAD = CHUNKS * 16


def _lane_sum(v, lanes):
    for d in (8, 4, 2, 1):
        v = v + v.at[lanes ^ d].get(mode="promise_in_bounds")
    return v


def _chunk_contrib(props_ref, tile_v, sem, lanes, itv, vv, tgc, valid, adv4n,
                   off):
    bv = lax.div(itv, jnp.int32(L))
    lv = lax.rem(itv, jnp.int32(L))
    tl8v = lax.bitwise_and(lv, jnp.int32(~7))
    tv128v = lax.bitwise_and(vv, jnp.int32(~127))
    rowv = lax.bitwise_and(lv, jnp.int32(7))
    cgv = lax.bitwise_and(vv, jnp.int32(112))
    lanev = lax.bitwise_and(vv, jnp.int32(15))

    copies = []
    for j in range(16):
        src = props_ref.at[bv[j], pl.ds(pl.multiple_of(tl8v[j], 8), 8),
                           pl.ds(pl.multiple_of(tv128v[j], 128), 128)]
        copies.append(pltpu.async_copy(src, tile_v.at[j], sem))
    for c in copies:
        c.wait()

    valsc = jnp.zeros((16,), jnp.float32)
    for j in range(16):
        v16 = tile_v[j, rowv[j], pl.ds(cgv[j], 16)]
        ev = v16.at[jnp.broadcast_to(lanev[j], (16,))].get(
            mode="promise_in_bounds")
        valsc = jnp.where(lanes == j, ev, valsc)

    m = jnp.where(valid, jnp.float32(1.0), jnp.float32(0.0))
    jvec = lax.div(off + lanes, jnp.int32(L))
    a = adv4n.at[jvec].get(mode="promise_in_bounds")
    return valsc * m * a, m


def _body(props_ref, sw_ref, tgt_ref, adv_ref, out_ref,
          sw_v, tg_v, adv_v, part_v, red_v, out_v, shared, sem, tile_v):
    cid = lax.axis_index("c")
    sid = lax.axis_index("s")
    wid = sid * NC + cid
    base = wid * PER_W

    pltpu.sync_copy(sw_ref.at[pl.ds(base, PER_W)], sw_v.at[pl.ds(0, PER_W)])
    pltpu.sync_copy(tgt_ref.at[pl.ds(base, PER_W)], tg_v.at[pl.ds(0, PER_W)])
    pltpu.sync_copy(adv_ref, adv_v)

    lanes = lax.iota(jnp.int32, 16)

    achunks = [adv_v[pl.ds(i * 16, 16)] for i in range(B // 16)]
    s = jnp.zeros((16,), jnp.float32)
    for c in achunks:
        s = s + c
    mean = _lane_sum(s, lanes) / jnp.float32(B)
    q = jnp.zeros((16,), jnp.float32)
    for c in achunks:
        d = c - mean
        q = q + d * d
    x = _lane_sum(q, lanes) / jnp.float32(B - 1)
    seed = jnp.int32(0x5F3759DF) - lax.shift_right_logical(
        lax.bitcast_convert_type(x, jnp.int32), 1)
    y = lax.bitcast_convert_type(seed, jnp.float32)
    for _ in range(4):
        y = y * (jnp.float32(1.5) - jnp.float32(0.5) * x * y * y)
    rstd = jnp.minimum(y, jnp.float32(1e8))

    start = wid * (PER_W // L)
    cidx = lax.div(start, jnp.int32(16))
    pos = lax.rem(start, jnp.int32(16))
    avec = adv_v[pl.ds(cidx * 16, 16)]
    perm = jnp.minimum(pos + lanes, jnp.int32(15))
    adv4n = (avec.at[perm].get(mode="promise_in_bounds") - mean) * rstd

    def chunk_body(k, carry):
        nacc, dacc = carry
        off = k * 16
        itv = base + off + lanes
        vv = sw_v[pl.ds(off, 16)]
        tgc = tg_v[pl.ds(off, 16)]
        contrib, m = _chunk_contrib(props_ref, tile_v, sem, lanes, itv, vv,
                                    tgc, tgc > 0, adv4n, off)
        return nacc - contrib, dacc + m

    nacc, dacc = lax.fori_loop(
        0, CHUNKS - 1, chunk_body,
        (jnp.zeros((16,), jnp.float32), jnp.zeros((16,), jnp.float32)))

    off = (CHUNKS - 1) * 16
    tail = lanes < (PER_W - off)
    itv = jnp.where(tail, base + off + lanes, base)
    vv = jnp.where(tail, sw_v[pl.ds(off, 16)], 0)
    tgc = tg_v[pl.ds(off, 16)]
    contrib, m = _chunk_contrib(props_ref, tile_v, sem, lanes, itv, vv,
                                tgc, (tgc > 0) & tail, adv4n, off)
    nacc = nacc - contrib
    dacc = dacc + m

    n_s = _lane_sum(nacc, lanes)
    d_s = _lane_sum(dacc, lanes)
    part_v[...] = jnp.where(lanes == 0, n_s,
                            jnp.where(lanes == 1, d_s, jnp.float32(0.0)))
    pltpu.sync_copy(part_v, shared.at[pl.ds(sid * 16, 16)])
    plsc.subcore_barrier()

    @pl.when(sid == 0)
    def _():
        pltpu.sync_copy(shared, red_v)
        tot = jnp.zeros((16,), jnp.float32)
        for i in range(NS):
            tot = tot + red_v[pl.ds(i * 16, 16)]
        out_v[...] = tot
        pltpu.sync_copy(out_v, out_ref.at[pl.ds(cid * 16, 16)])


_sc_call = functools.partial(
    pl.kernel,
    mesh=plsc.VectorSubcoreMesh(core_axis_name="c", subcore_axis_name="s"),
    out_type=jax.ShapeDtypeStruct((NC * 16,), jnp.float32),
    scratch_types=[
        pltpu.VMEM((PAD,), jnp.int32),
        pltpu.VMEM((PAD,), jnp.int32),
        pltpu.VMEM((B,), jnp.float32),
        pltpu.VMEM((16,), jnp.float32),
        pltpu.VMEM((NS * 16,), jnp.float32),
        pltpu.VMEM((16,), jnp.float32),
        pltpu.VMEM_SHARED((NS * 16,), jnp.float32),
        pltpu.SemaphoreType.DMA,
        pltpu.VMEM((16, 8, 128), jnp.float32),
    ],
)(_body)


def kernel(props, s_words, tgt, advantage):
    sw = s_words.astype(jnp.int32).reshape(-1)
    tg = tgt.astype(jnp.int32).reshape(-1)
    adv = advantage.astype(jnp.float32)
    out = _sc_call(props, sw, tg, adv)
    return (out[0] + out[16]) / (out[1] + out[17])


# speedup vs baseline: 55.3675x; 5.5551x over previous
import functools

import jax
import jax.numpy as jnp
from jax import lax
from jax.experimental import pallas as pl
from jax.experimental.pallas import tpu as pltpu
from jax.experimental.pallas import tpu_sc as plsc

B, L, V = 128, 50, 10000
N_ITEMS = B * L
NC, NS = 2, 16
NW = NC * NS
PER_W = N_ITEMS // NW
CHUNKS = (PER_W + 15) // 16
PAD = CHUNKS * 16


def _lane_sum(v, lanes):
    for d in (8, 4, 2, 1):
        v = v + v.at[lanes ^ d].get(mode="promise_in_bounds")
    return v


def _chunk_contrib(props_ref, tile_v, sem, lanes, itv, vv, tgc, valid, adv4n,
                   off):
    bv = lax.div(itv, jnp.int32(L))
    lv = lax.rem(itv, jnp.int32(L))
    tv8v = lax.bitwise_and(vv, jnp.int32(~7))
    rowv = lax.bitwise_and(vv, jnp.int32(7))
    cgv = lax.bitwise_and(bv, jnp.int32(112))
    lanev = lax.bitwise_and(bv, jnp.int32(15))

    copies = []
    for j in range(16):
        src = props_ref.at[lv[j], pl.ds(pl.multiple_of(tv8v[j], 8), 8),
                           pl.ds(0, 128)]
        copies.append(pltpu.async_copy(src, tile_v.at[j], sem))
    for c in copies:
        c.wait()

    valsc = jnp.zeros((16,), jnp.float32)
    for j in range(16):
        v16 = tile_v[j, rowv[j], pl.ds(cgv[j], 16)]
        ev = v16.at[jnp.broadcast_to(lanev[j], (16,))].get(
            mode="promise_in_bounds")
        valsc = jnp.where(lanes == j, ev, valsc)

    m = jnp.where(valid, jnp.float32(1.0), jnp.float32(0.0))
    jvec = lax.div(off + lanes, jnp.int32(L))
    a = adv4n.at[jvec].get(mode="promise_in_bounds")
    return valsc * m * a, m


def _body(props_ref, sw_ref, tgt_ref, adv_ref, out_ref,
          sw_v, tg_v, adv_v, part_v, red_v, out_v, shared, sem, tile_v):
    cid = lax.axis_index("c")
    sid = lax.axis_index("s")
    wid = sid * NC + cid
    base = wid * PER_W

    pltpu.sync_copy(sw_ref.at[pl.ds(base, PER_W)], sw_v.at[pl.ds(0, PER_W)])
    pltpu.sync_copy(tgt_ref.at[pl.ds(base, PER_W)], tg_v.at[pl.ds(0, PER_W)])
    pltpu.sync_copy(adv_ref, adv_v)

    lanes = lax.iota(jnp.int32, 16)

    achunks = [adv_v[pl.ds(i * 16, 16)] for i in range(B // 16)]
    s = jnp.zeros((16,), jnp.float32)
    for c in achunks:
        s = s + c
    mean = _lane_sum(s, lanes) / jnp.float32(B)
    q = jnp.zeros((16,), jnp.float32)
    for c in achunks:
        d = c - mean
        q = q + d * d
    x = _lane_sum(q, lanes) / jnp.float32(B - 1)
    seed = jnp.int32(0x5F3759DF) - lax.shift_right_logical(
        lax.bitcast_convert_type(x, jnp.int32), 1)
    y = lax.bitcast_convert_type(seed, jnp.float32)
    for _ in range(4):
        y = y * (jnp.float32(1.5) - jnp.float32(0.5) * x * y * y)
    rstd = jnp.minimum(y, jnp.float32(1e8))

    start = wid * (PER_W // L)
    cidx = lax.div(start, jnp.int32(16))
    pos = lax.rem(start, jnp.int32(16))
    avec = adv_v[pl.ds(cidx * 16, 16)]
    perm = jnp.minimum(pos + lanes, jnp.int32(15))
    adv4n = (avec.at[perm].get(mode="promise_in_bounds") - mean) * rstd

    def chunk_body(k, carry):
        nacc, dacc = carry
        off = k * 16
        itv = base + off + lanes
        vv = sw_v[pl.ds(off, 16)]
        tgc = tg_v[pl.ds(off, 16)]
        contrib, m = _chunk_contrib(props_ref, tile_v, sem, lanes, itv, vv,
                                    tgc, tgc > 0, adv4n, off)
        return nacc - contrib, dacc + m

    nacc, dacc = lax.fori_loop(
        0, CHUNKS - 1, chunk_body,
        (jnp.zeros((16,), jnp.float32), jnp.zeros((16,), jnp.float32)))

    off = (CHUNKS - 1) * 16
    tail = lanes < (PER_W - off)
    itv = jnp.where(tail, base + off + lanes, base)
    vv = jnp.where(tail, sw_v[pl.ds(off, 16)], 0)
    tgc = tg_v[pl.ds(off, 16)]
    contrib, m = _chunk_contrib(props_ref, tile_v, sem, lanes, itv, vv,
                                tgc, (tgc > 0) & tail, adv4n, off)
    nacc = nacc - contrib
    dacc = dacc + m

    n_s = _lane_sum(nacc, lanes)
    d_s = _lane_sum(dacc, lanes)
    part_v[...] = jnp.where(lanes == 0, n_s,
                            jnp.where(lanes == 1, d_s, jnp.float32(0.0)))
    pltpu.sync_copy(part_v, shared.at[pl.ds(sid * 16, 16)])
    plsc.subcore_barrier()

    @pl.when(sid == 0)
    def _():
        pltpu.sync_copy(shared, red_v)
        tot = jnp.zeros((16,), jnp.float32)
        for i in range(NS):
            tot = tot + red_v[pl.ds(i * 16, 16)]
        out_v[...] = tot
        pltpu.sync_copy(out_v, out_ref.at[pl.ds(cid * 16, 16)])


_sc_call = functools.partial(
    pl.kernel,
    mesh=plsc.VectorSubcoreMesh(core_axis_name="c", subcore_axis_name="s"),
    out_type=jax.ShapeDtypeStruct((NC * 16,), jnp.float32),
    scratch_types=[
        pltpu.VMEM((PAD,), jnp.int32),
        pltpu.VMEM((PAD,), jnp.int32),
        pltpu.VMEM((B,), jnp.float32),
        pltpu.VMEM((16,), jnp.float32),
        pltpu.VMEM((NS * 16,), jnp.float32),
        pltpu.VMEM((16,), jnp.float32),
        pltpu.VMEM_SHARED((NS * 16,), jnp.float32),
        pltpu.SemaphoreType.DMA,
        pltpu.VMEM((16, 8, 128), jnp.float32),
    ],
)(_body)


def kernel(props, s_words, tgt, advantage):
    pt = jnp.transpose(props, (1, 2, 0))
    sw = s_words.astype(jnp.int32).reshape(-1)
    tg = tgt.astype(jnp.int32).reshape(-1)
    adv = advantage.astype(jnp.float32)
    out = _sc_call(pt, sw, tg, adv)
    return (out[0] + out[16]) / (out[1] + out[17])


# speedup vs baseline: 78.0734x; 1.4101x over previous
import functools

import jax
import jax.numpy as jnp
from jax import lax
from jax.experimental import pallas as pl
from jax.experimental.pallas import tpu as pltpu
from jax.experimental.pallas import tpu_sc as plsc

B, L, V = 128, 50, 10000
N_ITEMS = B * L
NC, NS = 2, 16
NW = NC * NS
PER_W = N_ITEMS // NW
CHUNKS = (PER_W + 15) // 16
PAD = CHUNKS * 16


def _lane_sum(v, lanes):
    for d in (8, 4, 2, 1):
        v = v + v.at[lanes ^ d].get(mode="promise_in_bounds")
    return v


def _chunk_extract(rows_v, lanes, itv, off):
    bv = lax.div(itv, jnp.int32(L))
    cgv = lax.bitwise_and(bv, jnp.int32(112))
    lanev = lax.bitwise_and(bv, jnp.int32(15))
    valsc = jnp.zeros((16,), jnp.float32)
    for j in range(16):
        v16 = rows_v[off + j, pl.ds(cgv[j], 16)]
        ev = v16.at[jnp.broadcast_to(lanev[j], (16,))].get(
            mode="promise_in_bounds")
        valsc = jnp.where(lanes == j, ev, valsc)
    return valsc


def _body(props_ref, sw_ref, tgt_ref, adv_ref, out_ref,
          sw_v, tg_v, adv_v, part_v, red_v, out_v, shared, sem, rows_v):
    cid = lax.axis_index("c")
    sid = lax.axis_index("s")
    wid = sid * NC + cid
    base = wid * PER_W

    pltpu.sync_copy(sw_ref.at[pl.ds(base, PER_W)], sw_v.at[pl.ds(0, PER_W)])
    pltpu.sync_copy(tgt_ref.at[pl.ds(base, PER_W)], tg_v.at[pl.ds(0, PER_W)])
    pltpu.sync_copy(adv_ref, adv_v)

    lanes = lax.iota(jnp.int32, 16)

    achunks = [adv_v[pl.ds(i * 16, 16)] for i in range(B // 16)]
    s = jnp.zeros((16,), jnp.float32)
    for c in achunks:
        s = s + c
    mean = _lane_sum(s, lanes) / jnp.float32(B)
    q = jnp.zeros((16,), jnp.float32)
    for c in achunks:
        d = c - mean
        q = q + d * d
    x = _lane_sum(q, lanes) / jnp.float32(B - 1)
    seed = jnp.int32(0x5F3759DF) - lax.shift_right_logical(
        lax.bitcast_convert_type(x, jnp.int32), 1)
    y = lax.bitcast_convert_type(seed, jnp.float32)
    for _ in range(4):
        y = y * (jnp.float32(1.5) - jnp.float32(0.5) * x * y * y)
    rstd = jnp.minimum(y, jnp.float32(1e8))

    start = wid * (PER_W // L)
    cidx = lax.div(start, jnp.int32(16))
    pos = lax.rem(start, jnp.int32(16))
    avec = adv_v[pl.ds(cidx * 16, 16)]
    perm = jnp.minimum(pos + lanes, jnp.int32(15))
    adv4n = (avec.at[perm].get(mode="promise_in_bounds") - mean) * rstd

    copies = []
    for k in range(CHUNKS):
        off = k * 16
        itv = base + off + lanes
        vv = sw_v[pl.ds(off, 16)]
        if off + 16 > PER_W:
            tail = lanes < (PER_W - off)
            itv = jnp.where(tail, itv, base)
            vv = jnp.where(tail, vv, 0)
        rowidx = lax.rem(itv, jnp.int32(L)) * V + vv
        copies.append(pltpu.async_copy(props_ref.at[rowidx],
                                       rows_v.at[pl.ds(off, 16)], sem))
    for c in copies:
        c.wait()

    def chunk_body(k, carry):
        nacc, dacc = carry
        off = k * 16
        itv = base + off + lanes
        tgc = tg_v[pl.ds(off, 16)]
        m = jnp.where(tgc > 0, jnp.float32(1.0), jnp.float32(0.0))
        valsc = _chunk_extract(rows_v, lanes, itv, off)
        jvec = lax.div(off + lanes, jnp.int32(L))
        a = adv4n.at[jvec].get(mode="promise_in_bounds")
        return nacc - valsc * m * a, dacc + m

    nacc, dacc = lax.fori_loop(
        0, CHUNKS - 1, chunk_body,
        (jnp.zeros((16,), jnp.float32), jnp.zeros((16,), jnp.float32)))

    off = (CHUNKS - 1) * 16
    tail = lanes < (PER_W - off)
    itv = jnp.where(tail, base + off + lanes, base)
    tgc = tg_v[pl.ds(off, 16)]
    m = jnp.where((tgc > 0) & tail, jnp.float32(1.0), jnp.float32(0.0))
    valsc = _chunk_extract(rows_v, lanes, itv, off)
    jvec = lax.div(jnp.int32(off) + lanes, jnp.int32(L))
    a = adv4n.at[jvec].get(mode="promise_in_bounds")
    nacc = nacc - valsc * m * a
    dacc = dacc + m

    n_s = _lane_sum(nacc, lanes)
    d_s = _lane_sum(dacc, lanes)
    part_v[...] = jnp.where(lanes == 0, n_s,
                            jnp.where(lanes == 1, d_s, jnp.float32(0.0)))
    pltpu.sync_copy(part_v, shared.at[pl.ds(sid * 16, 16)])
    plsc.subcore_barrier()

    @pl.when(sid == 0)
    def _():
        pltpu.sync_copy(shared, red_v)
        tot = jnp.zeros((16,), jnp.float32)
        for i in range(NS):
            tot = tot + red_v[pl.ds(i * 16, 16)]
        out_v[...] = tot
        pltpu.sync_copy(out_v, out_ref.at[pl.ds(cid * 16, 16)])


_sc_call = functools.partial(
    pl.kernel,
    mesh=plsc.VectorSubcoreMesh(core_axis_name="c", subcore_axis_name="s"),
    out_type=jax.ShapeDtypeStruct((NC * 16,), jnp.float32),
    scratch_types=[
        pltpu.VMEM((PAD,), jnp.int32),
        pltpu.VMEM((PAD,), jnp.int32),
        pltpu.VMEM((B,), jnp.float32),
        pltpu.VMEM((16,), jnp.float32),
        pltpu.VMEM((NS * 16,), jnp.float32),
        pltpu.VMEM((16,), jnp.float32),
        pltpu.VMEM_SHARED((NS * 16,), jnp.float32),
        pltpu.SemaphoreType.DMA,
        pltpu.VMEM((PAD, 128), jnp.float32),
    ],
)(_body)


def kernel(props, s_words, tgt, advantage):
    pt = jnp.transpose(props, (1, 2, 0)).reshape(L * V, B)
    sw = s_words.astype(jnp.int32).reshape(-1)
    tg = tgt.astype(jnp.int32).reshape(-1)
    adv = advantage.astype(jnp.float32)
    out = _sc_call(pt, sw, tg, adv)
    return (out[0] + out[16]) / (out[1] + out[17])


# speedup vs baseline: 105.5266x; 1.3516x over previous
import functools

import jax
import jax.numpy as jnp
from jax import lax
from jax.experimental import pallas as pl
from jax.experimental.pallas import tpu as pltpu
from jax.experimental.pallas import tpu_sc as plsc

B, L, V = 128, 50, 10000
N_ITEMS = B * L
NC, NS = 2, 16
NW = NC * NS
PER_W = N_ITEMS // NW
CHUNKS = (PER_W + 15) // 16
PAD = CHUNKS * 16


def _lane_sum(v, lanes):
    for d in (8, 4, 2, 1):
        v = v + v.at[lanes ^ d].get(mode="promise_in_bounds")
    return v


def _body(props_ref, sw_ref, tgt_ref, adv_ref, out_ref,
          sw_v, tg_v, adv_v, part_v, red_v, out_v, shared, sem, vals_v):
    cid = lax.axis_index("c")
    sid = lax.axis_index("s")
    wid = sid * NC + cid
    base = wid * PER_W

    pltpu.sync_copy(sw_ref.at[pl.ds(base, PER_W)], sw_v.at[pl.ds(0, PER_W)])
    pltpu.sync_copy(tgt_ref.at[pl.ds(base, PER_W)], tg_v.at[pl.ds(0, PER_W)])
    pltpu.sync_copy(adv_ref, adv_v)

    lanes = lax.iota(jnp.int32, 16)

    achunks = [adv_v[pl.ds(i * 16, 16)] for i in range(B // 16)]
    s = jnp.zeros((16,), jnp.float32)
    for c in achunks:
        s = s + c
    mean = _lane_sum(s, lanes) / jnp.float32(B)
    q = jnp.zeros((16,), jnp.float32)
    for c in achunks:
        d = c - mean
        q = q + d * d
    x = _lane_sum(q, lanes) / jnp.float32(B - 1)
    seed = jnp.int32(0x5F3759DF) - lax.shift_right_logical(
        lax.bitcast_convert_type(x, jnp.int32), 1)
    y = lax.bitcast_convert_type(seed, jnp.float32)
    for _ in range(4):
        y = y * (jnp.float32(1.5) - jnp.float32(0.5) * x * y * y)
    rstd = jnp.minimum(y, jnp.float32(1e8))

    start = wid * (PER_W // L)
    cidx = lax.div(start, jnp.int32(16))
    pos = lax.rem(start, jnp.int32(16))
    avec = adv_v[pl.ds(cidx * 16, 16)]
    perm = jnp.minimum(pos + lanes, jnp.int32(15))
    adv4n = (avec.at[perm].get(mode="promise_in_bounds") - mean) * rstd

    copies = []
    for k in range(CHUNKS):
        off = k * 16
        itv = base + off + lanes
        vv = sw_v[pl.ds(off, 16)]
        if off + 16 > PER_W:
            tail = lanes < (PER_W - off)
            itv = jnp.where(tail, itv, base)
            vv = jnp.where(tail, vv, 0)
        bv = lax.div(itv, jnp.int32(L))
        lv = lax.rem(itv, jnp.int32(L))
        flat = lax.shift_left(lv * V + vv, 7) + bv
        copies.append(pltpu.async_copy(props_ref.at[flat],
                                       vals_v.at[k], sem))
    for c in copies:
        c.wait()

    def chunk_body(k, carry):
        nacc, dacc = carry
        off = k * 16
        tgc = tg_v[pl.ds(off, 16)]
        m = jnp.where(tgc > 0, jnp.float32(1.0), jnp.float32(0.0))
        jvec = lax.div(off + lanes, jnp.int32(L))
        a = adv4n.at[jvec].get(mode="promise_in_bounds")
        return nacc - vals_v[k] * m * a, dacc + m

    nacc, dacc = lax.fori_loop(
        0, CHUNKS - 1, chunk_body,
        (jnp.zeros((16,), jnp.float32), jnp.zeros((16,), jnp.float32)))

    off = (CHUNKS - 1) * 16
    tail = lanes < (PER_W - off)
    tgc = tg_v[pl.ds(off, 16)]
    m = jnp.where((tgc > 0) & tail, jnp.float32(1.0), jnp.float32(0.0))
    jvec = lax.div(jnp.int32(off) + lanes, jnp.int32(L))
    a = adv4n.at[jvec].get(mode="promise_in_bounds")
    nacc = nacc - vals_v[CHUNKS - 1] * m * a
    dacc = dacc + m

    n_s = _lane_sum(nacc, lanes)
    d_s = _lane_sum(dacc, lanes)
    part_v[...] = jnp.where(lanes == 0, n_s,
                            jnp.where(lanes == 1, d_s, jnp.float32(0.0)))
    pltpu.sync_copy(part_v, shared.at[pl.ds(sid * 16, 16)])
    plsc.subcore_barrier()

    @pl.when(sid == 0)
    def _():
        pltpu.sync_copy(shared, red_v)
        tot = jnp.zeros((16,), jnp.float32)
        for i in range(NS):
            tot = tot + red_v[pl.ds(i * 16, 16)]
        out_v[...] = tot
        pltpu.sync_copy(out_v, out_ref.at[pl.ds(cid * 16, 16)])


_sc_call = functools.partial(
    pl.kernel,
    mesh=plsc.VectorSubcoreMesh(core_axis_name="c", subcore_axis_name="s"),
    out_type=jax.ShapeDtypeStruct((NC * 16,), jnp.float32),
    scratch_types=[
        pltpu.VMEM((PAD,), jnp.int32),
        pltpu.VMEM((PAD,), jnp.int32),
        pltpu.VMEM((B,), jnp.float32),
        pltpu.VMEM((16,), jnp.float32),
        pltpu.VMEM((NS * 16,), jnp.float32),
        pltpu.VMEM((16,), jnp.float32),
        pltpu.VMEM_SHARED((NS * 16,), jnp.float32),
        pltpu.SemaphoreType.DMA,
        pltpu.VMEM((CHUNKS, 16), jnp.float32),
    ],
)(_body)


def kernel(props, s_words, tgt, advantage):
    pt = jnp.transpose(props, (1, 2, 0)).reshape(-1)
    sw = s_words.astype(jnp.int32).reshape(-1)
    tg = tgt.astype(jnp.int32).reshape(-1)
    adv = advantage.astype(jnp.float32)
    out = _sc_call(pt, sw, tg, adv)
    return (out[0] + out[16]) / (out[1] + out[17])


# speedup vs baseline: 105.7638x; 1.0022x over previous
import functools

import jax
import jax.numpy as jnp
from jax import lax
from jax.experimental import pallas as pl
from jax.experimental.pallas import tpu as pltpu
from jax.experimental.pallas import tpu_sc as plsc

B, L, V = 128, 50, 10000
NC, NS = 2, 16
NGROUPS = (L + 7) // 8
NQ = 4
ACTIVE = NGROUPS * NQ


def _lane_sum(v, lanes):
    for d in (8, 4, 2, 1):
        v = v + v.at[lanes ^ d].get(mode="promise_in_bounds")
    return v


def _body(props_ref, sw_ref, tgt_ref, adv_ref, out_ref,
          sw_g, tg_g, adv_v, part_v, red_v, out_v, shared, sem, vals_v):
    cid = lax.axis_index("c")
    sid = lax.axis_index("s")
    wid = sid * NC + cid
    g = jnp.minimum(lax.shift_right_logical(wid, 2), jnp.int32(NGROUPS - 1))
    q = lax.bitwise_and(wid, jnp.int32(3))
    wvalid = jnp.minimum(jnp.maximum(
        jnp.float32(ACTIVE) - wid.astype(jnp.float32), jnp.float32(0.0)),
        jnp.float32(1.0))

    g8 = g * 8
    pltpu.sync_copy(sw_ref.at[pl.ds(pl.multiple_of(g8, 8), 8), :], sw_g)
    pltpu.sync_copy(tgt_ref.at[pl.ds(pl.multiple_of(g8, 8), 8), :], tg_g)
    pltpu.sync_copy(adv_ref, adv_v)

    lanes = lax.iota(jnp.int32, 16)

    achunks = [adv_v[pl.ds(i * 16, 16)] for i in range(B // 16)]
    s = jnp.zeros((16,), jnp.float32)
    for c in achunks:
        s = s + c
    mean = _lane_sum(s, lanes) / jnp.float32(B)
    qq = jnp.zeros((16,), jnp.float32)
    for c in achunks:
        d = c - mean
        qq = qq + d * d
    x = _lane_sum(qq, lanes) / jnp.float32(B - 1)
    seed = jnp.int32(0x5F3759DF) - lax.shift_right_logical(
        lax.bitcast_convert_type(x, jnp.int32), 1)
    y = lax.bitcast_convert_type(seed, jnp.float32)
    for _ in range(4):
        y = y * (jnp.float32(1.5) - jnp.float32(0.5) * x * y * y)
    rstd = jnp.minimum(y, jnp.float32(1e8))
    for i in range(B // 16):
        adv_v[pl.ds(i * 16, 16)] = (achunks[i] - mean) * rstd

    copies = []
    for r in range(8):
        lrow = g8 + r
        lc = jnp.minimum(lrow, jnp.int32(L - 1))
        for t in range(2):
            boff = q * 32 + t * 16
            vv = sw_g[r, pl.ds(boff, 16)]
            vv = jnp.maximum(jnp.minimum(vv, jnp.int32(V - 1)), jnp.int32(0))
            flat = lax.shift_left(lc * V + vv, 7) + boff + lanes
            copies.append(pltpu.async_copy(props_ref.at[flat],
                                           vals_v.at[2 * r + t], sem))
    for c in copies:
        c.wait()

    nacc = jnp.zeros((16,), jnp.float32)
    dacc = jnp.zeros((16,), jnp.float32)
    for r in range(8):
        lrow = g8 + r
        rvalid = jnp.minimum(jnp.maximum(
            jnp.float32(L) - lrow.astype(jnp.float32), jnp.float32(0.0)),
            jnp.float32(1.0)) * wvalid
        rv = jnp.broadcast_to(rvalid, (16,))
        for t in range(2):
            boff = q * 32 + t * 16
            tgc = tg_g[r, pl.ds(boff, 16)]
            m = jnp.where(tgc > 0, jnp.float32(1.0), jnp.float32(0.0)) * rv
            a = adv_v[pl.ds(boff, 16)]
            nacc = nacc - vals_v[2 * r + t] * m * a
            dacc = dacc + m

    n_s = _lane_sum(nacc, lanes)
    d_s = _lane_sum(dacc, lanes)
    part_v[...] = jnp.where(lanes == 0, n_s,
                            jnp.where(lanes == 1, d_s, jnp.float32(0.0)))
    pltpu.sync_copy(part_v, shared.at[pl.ds(sid * 16, 16)])
    plsc.subcore_barrier()

    @pl.when(sid == 0)
    def _():
        pltpu.sync_copy(shared, red_v)
        tot = jnp.zeros((16,), jnp.float32)
        for i in range(NS):
            tot = tot + red_v[pl.ds(i * 16, 16)]
        out_v[...] = tot
        pltpu.sync_copy(out_v, out_ref.at[pl.ds(cid * 16, 16)])


_sc_call = functools.partial(
    pl.kernel,
    mesh=plsc.VectorSubcoreMesh(core_axis_name="c", subcore_axis_name="s"),
    out_type=jax.ShapeDtypeStruct((NC * 16,), jnp.float32),
    scratch_types=[
        pltpu.VMEM((8, B), jnp.int32),
        pltpu.VMEM((8, B), jnp.int32),
        pltpu.VMEM((B,), jnp.float32),
        pltpu.VMEM((16,), jnp.float32),
        pltpu.VMEM((NS * 16,), jnp.float32),
        pltpu.VMEM((16,), jnp.float32),
        pltpu.VMEM_SHARED((NS * 16,), jnp.float32),
        pltpu.SemaphoreType.DMA,
        pltpu.VMEM((16, 16), jnp.float32),
    ],
)(_body)


def kernel(props, s_words, tgt, advantage):
    pt = jnp.transpose(props, (1, 2, 0)).reshape(-1)
    sw = jnp.transpose(s_words.astype(jnp.int32), (1, 0))
    tg = jnp.transpose(tgt.astype(jnp.int32), (1, 0))
    adv = advantage.astype(jnp.float32)
    out = _sc_call(pt, sw, tg, adv)
    return (out[0] + out[16]) / (out[1] + out[17])
